# Initial kernel scaffold; baseline (speedup 1.0000x reference)
import jax
import jax.numpy as jnp
from jax.experimental import pallas as pl


def kernel(activations, sample_indices):
    raise NotImplementedError("write your pallas kernel here")


# speedup vs baseline: 1.1935x; 1.1935x over previous
import functools

import jax
import jax.numpy as jnp
from jax import lax
from jax.experimental import pallas as pl
from jax.experimental.pallas import tpu as pltpu, tpu_sc as plsc

B = 256
N_IN = 65536
N_OUT = 65536

NC = 2
NS = 16
NW = NC * NS
L = 16

ROWS_PER_W = B // NW
CHUNK = 16384
N_CHUNKS = N_OUT // CHUNK


def _sampler_body(act_hbm, pidx_hbm, out_hbm, row_buf, idx_buf, out_buf):
    wid = lax.axis_index("s") * NC + lax.axis_index("c")
    row0 = wid * ROWS_PER_W

    def row_body(r, carry):
        row = row0 + r
        pltpu.sync_copy(act_hbm.at[row], row_buf)

        def chunk_body(c, carry):
            base = pl.multiple_of(c * CHUNK, CHUNK)
            pltpu.sync_copy(pidx_hbm.at[pl.ds(base, CHUNK)], idx_buf)

            def grp(j, carry):
                off = pl.multiple_of(j * L, L)
                packed = idx_buf[pl.ds(off, L)]
                i0 = lax.bitwise_and(packed, jnp.int32(0xFFFF))
                i1 = lax.shift_right_logical(packed, jnp.int32(16))
                s0 = plsc.load_gather(row_buf, [i0])
                s1 = plsc.load_gather(row_buf, [i1])
                out_buf[pl.ds(off, L)] = 1.0 - s0 * s1
                return carry

            lax.fori_loop(0, CHUNK // L, grp, 0, unroll=4)
            pltpu.sync_copy(out_buf, out_hbm.at[row, pl.ds(base, CHUNK)])
            return carry

        lax.fori_loop(0, N_CHUNKS, chunk_body, 0)
        return carry

    lax.fori_loop(0, ROWS_PER_W, row_body, 0)


@functools.partial(
    pl.kernel,
    out_type=jax.ShapeDtypeStruct((B, N_OUT), jnp.float32),
    mesh=plsc.VectorSubcoreMesh(core_axis_name="c", subcore_axis_name="s"),
    scratch_types=[
        pltpu.VMEM((N_IN,), jnp.float32),
        pltpu.VMEM((CHUNK,), jnp.int32),
        pltpu.VMEM((CHUNK,), jnp.float32),
    ],
    compiler_params=pltpu.CompilerParams(needs_layout_passes=False),
)
def _sampler_kernel(act_hbm, pidx_hbm, out_hbm, row_buf, idx_buf, out_buf):
    _sampler_body(act_hbm, pidx_hbm, out_hbm, row_buf, idx_buf, out_buf)


def kernel(activations, sample_indices):
    idx = sample_indices.astype(jnp.int32)
    packed = jnp.bitwise_or(idx[:, 0], jnp.left_shift(idx[:, 1], 16))
    return _sampler_kernel(activations, packed)


# speedup vs baseline: 3.3788x; 2.8309x over previous
import functools

import jax
import jax.numpy as jnp
from jax import lax
from jax.experimental import pallas as pl
from jax.experimental.pallas import tpu as pltpu, tpu_sc as plsc

B = 256
N_IN = 65536
N_OUT = 65536

NC = 2
NS = 16
NW = NC * NS
L = 16

ROWS_PER_W = B // NW
CHUNK = 16384
N_CHUNKS = N_OUT // CHUNK


def _sampler_body(act_hbm, pidx_hbm, out_hbm, row_buf, idx_buf, out_buf):
    wid = lax.axis_index("s") * NC + lax.axis_index("c")
    row0 = wid * ROWS_PER_W

    def row_body(r, carry):
        row = row0 + r
        pltpu.sync_copy(act_hbm.at[row], row_buf)

        def chunk_body(c, carry):
            base = pl.multiple_of(c * CHUNK, CHUNK)
            pltpu.sync_copy(pidx_hbm.at[pl.ds(base, CHUNK)], idx_buf)

            @plsc.parallel_loop(0, CHUNK // L, unroll=8)
            def grp(j):
                off = pl.multiple_of(j * L, L)
                packed = idx_buf[pl.ds(off, L)]
                i0 = lax.bitwise_and(packed, jnp.int32(0xFFFF))
                i1 = lax.shift_right_logical(packed, jnp.int32(16))
                s0 = plsc.load_gather(row_buf, [i0])
                s1 = plsc.load_gather(row_buf, [i1])
                out_buf[pl.ds(off, L)] = 1.0 - s0 * s1
            pltpu.sync_copy(out_buf, out_hbm.at[row, pl.ds(base, CHUNK)])
            return carry

        lax.fori_loop(0, N_CHUNKS, chunk_body, 0)
        return carry

    lax.fori_loop(0, ROWS_PER_W, row_body, 0)


@functools.partial(
    pl.kernel,
    out_type=jax.ShapeDtypeStruct((B, N_OUT), jnp.float32),
    mesh=plsc.VectorSubcoreMesh(core_axis_name="c", subcore_axis_name="s"),
    scratch_types=[
        pltpu.VMEM((N_IN,), jnp.float32),
        pltpu.VMEM((CHUNK,), jnp.int32),
        pltpu.VMEM((CHUNK,), jnp.float32),
    ],
    compiler_params=pltpu.CompilerParams(needs_layout_passes=False),
)
def _sampler_kernel(act_hbm, pidx_hbm, out_hbm, row_buf, idx_buf, out_buf):
    _sampler_body(act_hbm, pidx_hbm, out_hbm, row_buf, idx_buf, out_buf)


def kernel(activations, sample_indices):
    idx = sample_indices.astype(jnp.int32)
    packed = jnp.bitwise_or(idx[:, 0], jnp.left_shift(idx[:, 1], 16))
    return _sampler_kernel(activations, packed)


# speedup vs baseline: 4.3352x; 1.2831x over previous
import functools

import jax
import jax.numpy as jnp
from jax import lax
from jax.experimental import pallas as pl
from jax.experimental.pallas import tpu as pltpu, tpu_sc as plsc

B = 256
N_IN = 65536
N_OUT = 65536

NC = 2
NS = 16
NW = NC * NS
L = 16

ROWS_PER_W = B // NW
CHUNK = 8192
N_CHUNKS = N_OUT // CHUNK


def _sampler_body(act_hbm, pidx_hbm, out_hbm, row_buf,
                  idx_bufs, out_bufs, sem_act, idx_sems, out_sems):
    wid = lax.axis_index("s") * NC + lax.axis_index("c")
    row0 = wid * ROWS_PER_W

    def compute_chunk(idx_buf, out_buf):
        @plsc.parallel_loop(0, CHUNK // L, unroll=8)
        def grp(j):
            off = pl.multiple_of(j * L, L)
            packed = idx_buf[pl.ds(off, L)]
            i0 = lax.bitwise_and(packed, jnp.int32(0xFFFF))
            i1 = lax.shift_right_logical(packed, jnp.int32(16))
            s0 = plsc.load_gather(row_buf, [i0])
            s1 = plsc.load_gather(row_buf, [i1])
            out_buf[pl.ds(off, L)] = 1.0 - s0 * s1

    def row_body(r, carry):
        row = row0 + r
        act_h = pltpu.async_copy(act_hbm.at[row], row_buf, sem_act)
        idx_h = [None] * N_CHUNKS
        out_h = [None] * N_CHUNKS
        idx_h[0] = pltpu.async_copy(
            pidx_hbm.at[pl.ds(0, CHUNK)], idx_bufs[0], idx_sems[0])
        act_h.wait()
        for c in range(N_CHUNKS):
            p = c % 2
            idx_h[c].wait()
            if c + 1 < N_CHUNKS:
                idx_h[c + 1] = pltpu.async_copy(
                    pidx_hbm.at[pl.ds((c + 1) * CHUNK, CHUNK)],
                    idx_bufs[1 - p], idx_sems[1 - p])
            if c >= 2:
                out_h[c - 2].wait()
            compute_chunk(idx_bufs[p], out_bufs[p])
            out_h[c] = pltpu.async_copy(
                out_bufs[p], out_hbm.at[row, pl.ds(c * CHUNK, CHUNK)],
                out_sems[p])
        out_h[N_CHUNKS - 2].wait()
        out_h[N_CHUNKS - 1].wait()
        return carry

    lax.fori_loop(0, ROWS_PER_W, row_body, 0)


@functools.partial(
    pl.kernel,
    out_type=jax.ShapeDtypeStruct((B, N_OUT), jnp.float32),
    mesh=plsc.VectorSubcoreMesh(core_axis_name="c", subcore_axis_name="s"),
    scratch_types=[
        pltpu.VMEM((N_IN,), jnp.float32),
        pltpu.VMEM((CHUNK,), jnp.int32),
        pltpu.VMEM((CHUNK,), jnp.int32),
        pltpu.VMEM((CHUNK,), jnp.float32),
        pltpu.VMEM((CHUNK,), jnp.float32),
        pltpu.SemaphoreType.DMA,
        pltpu.SemaphoreType.DMA,
        pltpu.SemaphoreType.DMA,
        pltpu.SemaphoreType.DMA,
        pltpu.SemaphoreType.DMA,
    ],
    compiler_params=pltpu.CompilerParams(needs_layout_passes=False),
)
def _sampler_kernel(act_hbm, pidx_hbm, out_hbm, row_buf,
                    idx_a, idx_b, out_a, out_b,
                    sem_act, sem_ia, sem_ib, sem_oa, sem_ob):
    _sampler_body(act_hbm, pidx_hbm, out_hbm, row_buf,
                  (idx_a, idx_b), (out_a, out_b),
                  sem_act, (sem_ia, sem_ib), (sem_oa, sem_ob))


def kernel(activations, sample_indices):
    idx = sample_indices.astype(jnp.int32)
    packed = jnp.bitwise_or(idx[:, 0], jnp.left_shift(idx[:, 1], 16))
    return _sampler_kernel(activations, packed)


# speedup vs baseline: 4.4479x; 1.0260x over previous
import functools

import jax
import jax.numpy as jnp
from jax import lax
from jax.experimental import pallas as pl
from jax.experimental.pallas import tpu as pltpu, tpu_sc as plsc

B = 256
N_IN = 65536
N_OUT = 65536

NC = 2
NS = 16
NW = NC * NS
L = 16

ROWS_PER_W = B // NW
CHUNK = 2048
N_CHUNKS = N_OUT // CHUNK
IDX_RES = 53248
RES_CHUNKS = IDX_RES // CHUNK
STR_CHUNKS = N_CHUNKS - RES_CHUNKS


def _sampler_body(act_hbm, pidx_hbm, out_hbm, row_buf, idx_res,
                  idx_bufs, out_bufs, sem_act, idx_sems, out_sems):
    wid = lax.axis_index("s") * NC + lax.axis_index("c")
    row0 = wid * ROWS_PER_W

    def compute_chunk(idx_buf, idx_off, out_buf):
        @plsc.parallel_loop(0, CHUNK // L, unroll=8)
        def grp(j):
            off = pl.multiple_of(j * L, L)
            packed = idx_buf[pl.ds(idx_off + off, L)]
            i0 = lax.bitwise_and(packed, jnp.int32(0xFFFF))
            i1 = lax.shift_right_logical(packed, jnp.int32(16))
            s0 = plsc.load_gather(row_buf, [i0])
            s1 = plsc.load_gather(row_buf, [i1])
            out_buf[pl.ds(off, L)] = 1.0 - s0 * s1

    pltpu.sync_copy(pidx_hbm.at[pl.ds(0, IDX_RES)], idx_res)

    def row_body(r, carry):
        row = row0 + r
        act_h = pltpu.async_copy(act_hbm.at[row], row_buf, sem_act)
        idx_h = [None] * STR_CHUNKS
        for k in range(min(2, STR_CHUNKS)):
            idx_h[k] = pltpu.async_copy(
                pidx_hbm.at[pl.ds(IDX_RES + k * CHUNK, CHUNK)],
                idx_bufs[k % 2], idx_sems[k % 2])
        act_h.wait()
        out_h = [None] * N_CHUNKS
        for c in range(N_CHUNKS):
            p = c % 2
            if c >= 2:
                out_h[c - 2].wait()
            if c < RES_CHUNKS:
                compute_chunk(idx_res, c * CHUNK, out_bufs[p])
            else:
                k = c - RES_CHUNKS
                idx_h[k].wait()
                compute_chunk(idx_bufs[k % 2], 0, out_bufs[p])
                if k + 2 < STR_CHUNKS:
                    idx_h[k + 2] = pltpu.async_copy(
                        pidx_hbm.at[pl.ds(IDX_RES + (k + 2) * CHUNK, CHUNK)],
                        idx_bufs[k % 2], idx_sems[k % 2])
            out_h[c] = pltpu.async_copy(
                out_bufs[p], out_hbm.at[row, pl.ds(c * CHUNK, CHUNK)],
                out_sems[p])
        out_h[N_CHUNKS - 2].wait()
        out_h[N_CHUNKS - 1].wait()
        return carry

    lax.fori_loop(0, ROWS_PER_W, row_body, 0)


@functools.partial(
    pl.kernel,
    out_type=jax.ShapeDtypeStruct((B, N_OUT), jnp.float32),
    mesh=plsc.VectorSubcoreMesh(core_axis_name="c", subcore_axis_name="s"),
    scratch_types=[
        pltpu.VMEM((N_IN,), jnp.float32),
        pltpu.VMEM((IDX_RES,), jnp.int32),
        pltpu.VMEM((CHUNK,), jnp.int32),
        pltpu.VMEM((CHUNK,), jnp.int32),
        pltpu.VMEM((CHUNK,), jnp.float32),
        pltpu.VMEM((CHUNK,), jnp.float32),
        pltpu.SemaphoreType.DMA,
        pltpu.SemaphoreType.DMA,
        pltpu.SemaphoreType.DMA,
        pltpu.SemaphoreType.DMA,
        pltpu.SemaphoreType.DMA,
    ],
    compiler_params=pltpu.CompilerParams(needs_layout_passes=False),
)
def _sampler_kernel(act_hbm, pidx_hbm, out_hbm, row_buf, idx_res,
                    idx_a, idx_b, out_a, out_b,
                    sem_act, sem_ia, sem_ib, sem_oa, sem_ob):
    _sampler_body(act_hbm, pidx_hbm, out_hbm, row_buf, idx_res,
                  (idx_a, idx_b), (out_a, out_b),
                  sem_act, (sem_ia, sem_ib), (sem_oa, sem_ob))


def kernel(activations, sample_indices):
    idx = sample_indices.astype(jnp.int32)
    packed = jnp.bitwise_or(idx[:, 0], jnp.left_shift(idx[:, 1], 16))
    return _sampler_kernel(activations, packed)


# speedup vs baseline: 4.4505x; 1.0006x over previous
import functools

import jax
import jax.numpy as jnp
from jax import lax
from jax.experimental import pallas as pl
from jax.experimental.pallas import tpu as pltpu, tpu_sc as plsc

B = 256
N_IN = 65536
N_OUT = 65536

NC = 2
NS = 16
NW = NC * NS
L = 16

ROWS_PER_W = B // NW
CHUNK = 8192
N_CHUNKS = N_OUT // CHUNK
DEPTH = 3


def _sampler_body(act_hbm, pidx_hbm, out_hbm, row_buf,
                  idx_bufs, out_bufs, sem_act, idx_sems, out_sems):
    wid = lax.axis_index("s") * NC + lax.axis_index("c")
    row0 = wid * ROWS_PER_W

    def compute_chunk(idx_buf, out_buf):
        @plsc.parallel_loop(0, CHUNK // L, unroll=8)
        def grp(j):
            off = pl.multiple_of(j * L, L)
            packed = idx_buf[pl.ds(off, L)]
            i0 = lax.bitwise_and(packed, jnp.int32(0xFFFF))
            i1 = lax.shift_right_logical(packed, jnp.int32(16))
            s0 = plsc.load_gather(row_buf, [i0])
            s1 = plsc.load_gather(row_buf, [i1])
            out_buf[pl.ds(off, L)] = 1.0 - s0 * s1

    def row_body(r, carry):
        row = row0 + r
        act_h = pltpu.async_copy(act_hbm.at[row], row_buf, sem_act)
        idx_h = [None] * N_CHUNKS
        out_h = [None] * N_CHUNKS
        for k in range(DEPTH):
            idx_h[k] = pltpu.async_copy(
                pidx_hbm.at[pl.ds(k * CHUNK, CHUNK)],
                idx_bufs[k % DEPTH], idx_sems[k % DEPTH])
        act_h.wait()
        for c in range(N_CHUNKS):
            p = c % DEPTH
            idx_h[c].wait()
            if c >= DEPTH:
                out_h[c - DEPTH].wait()
            compute_chunk(idx_bufs[p], out_bufs[p])
            if c + DEPTH < N_CHUNKS:
                idx_h[c + DEPTH] = pltpu.async_copy(
                    pidx_hbm.at[pl.ds((c + DEPTH) * CHUNK, CHUNK)],
                    idx_bufs[p], idx_sems[p])
            out_h[c] = pltpu.async_copy(
                out_bufs[p], out_hbm.at[row, pl.ds(c * CHUNK, CHUNK)],
                out_sems[p])
        for c in range(N_CHUNKS - DEPTH, N_CHUNKS):
            out_h[c].wait()
        return carry

    lax.fori_loop(0, ROWS_PER_W, row_body, 0)


@functools.partial(
    pl.kernel,
    out_type=jax.ShapeDtypeStruct((B, N_OUT), jnp.float32),
    mesh=plsc.VectorSubcoreMesh(core_axis_name="c", subcore_axis_name="s"),
    scratch_types=[
        pltpu.VMEM((N_IN,), jnp.float32),
        pltpu.VMEM((CHUNK,), jnp.int32),
        pltpu.VMEM((CHUNK,), jnp.int32),
        pltpu.VMEM((CHUNK,), jnp.int32),
        pltpu.VMEM((CHUNK,), jnp.float32),
        pltpu.VMEM((CHUNK,), jnp.float32),
        pltpu.VMEM((CHUNK,), jnp.float32),
        pltpu.SemaphoreType.DMA,
        pltpu.SemaphoreType.DMA,
        pltpu.SemaphoreType.DMA,
        pltpu.SemaphoreType.DMA,
        pltpu.SemaphoreType.DMA,
        pltpu.SemaphoreType.DMA,
        pltpu.SemaphoreType.DMA,
    ],
    compiler_params=pltpu.CompilerParams(needs_layout_passes=False),
)
def _sampler_kernel(act_hbm, pidx_hbm, out_hbm, row_buf,
                    idx_a, idx_b, idx_c, out_a, out_b, out_c,
                    sem_act, sem_i0, sem_i1, sem_i2,
                    sem_o0, sem_o1, sem_o2):
    _sampler_body(act_hbm, pidx_hbm, out_hbm, row_buf,
                  (idx_a, idx_b, idx_c), (out_a, out_b, out_c),
                  sem_act, (sem_i0, sem_i1, sem_i2),
                  (sem_o0, sem_o1, sem_o2))


def kernel(activations, sample_indices):
    idx = sample_indices.astype(jnp.int32)
    packed = jnp.bitwise_or(idx[:, 0], jnp.left_shift(idx[:, 1], 16))
    return _sampler_kernel(activations, packed)


# speedup vs baseline: 5.3145x; 1.1941x over previous
import functools

import jax
import jax.numpy as jnp
from jax import lax
from jax.experimental import pallas as pl
from jax.experimental.pallas import tpu as pltpu, tpu_sc as plsc

B = 256
N_IN = 65536
N_OUT = 65536

NC = 2
NS = 16
NW = NC * NS
L = 16

ROWS_PER_W = B // NW
CHUNK = 8192
N_CHUNKS = N_OUT // CHUNK
DEPTH = 3


def _sampler_body(act_hbm, pidx_hbm, out_hbm, row_buf, idx_shared,
                  idx_bufs, out_bufs, sem_act, idx_sems, out_sems):
    sid = lax.axis_index("s")
    wid = sid * NC + lax.axis_index("c")
    row0 = wid * ROWS_PER_W

    @pl.when(sid == 0)
    def _():
        pltpu.sync_copy(pidx_hbm, idx_shared)
    plsc.subcore_barrier()

    def compute_chunk(idx_buf, out_buf):
        @plsc.parallel_loop(0, CHUNK // L, unroll=8)
        def grp(j):
            off = pl.multiple_of(j * L, L)
            packed = idx_buf[pl.ds(off, L)]
            i0 = lax.bitwise_and(packed, jnp.int32(0xFFFF))
            i1 = lax.shift_right_logical(packed, jnp.int32(16))
            s0 = plsc.load_gather(row_buf, [i0])
            s1 = plsc.load_gather(row_buf, [i1])
            out_buf[pl.ds(off, L)] = 1.0 - s0 * s1

    def row_body(r, carry):
        row = row0 + r
        act_h = pltpu.async_copy(act_hbm.at[row], row_buf, sem_act)
        idx_h = [None] * N_CHUNKS
        out_h = [None] * N_CHUNKS
        for k in range(DEPTH):
            idx_h[k] = pltpu.async_copy(
                idx_shared.at[pl.ds(k * CHUNK, CHUNK)],
                idx_bufs[k % DEPTH], idx_sems[k % DEPTH])
        act_h.wait()
        for c in range(N_CHUNKS):
            p = c % DEPTH
            idx_h[c].wait()
            if c >= DEPTH:
                out_h[c - DEPTH].wait()
            compute_chunk(idx_bufs[p], out_bufs[p])
            if c + DEPTH < N_CHUNKS:
                idx_h[c + DEPTH] = pltpu.async_copy(
                    idx_shared.at[pl.ds((c + DEPTH) * CHUNK, CHUNK)],
                    idx_bufs[p], idx_sems[p])
            out_h[c] = pltpu.async_copy(
                out_bufs[p], out_hbm.at[row, pl.ds(c * CHUNK, CHUNK)],
                out_sems[p])
        for c in range(N_CHUNKS - DEPTH, N_CHUNKS):
            out_h[c].wait()
        return carry

    lax.fori_loop(0, ROWS_PER_W, row_body, 0)


@functools.partial(
    pl.kernel,
    out_type=jax.ShapeDtypeStruct((B, N_OUT), jnp.float32),
    mesh=plsc.VectorSubcoreMesh(core_axis_name="c", subcore_axis_name="s"),
    scratch_types=[
        pltpu.VMEM((N_IN,), jnp.float32),
        pltpu.VMEM_SHARED((N_OUT,), jnp.int32),
        pltpu.VMEM((CHUNK,), jnp.int32),
        pltpu.VMEM((CHUNK,), jnp.int32),
        pltpu.VMEM((CHUNK,), jnp.int32),
        pltpu.VMEM((CHUNK,), jnp.float32),
        pltpu.VMEM((CHUNK,), jnp.float32),
        pltpu.VMEM((CHUNK,), jnp.float32),
        pltpu.SemaphoreType.DMA,
        pltpu.SemaphoreType.DMA,
        pltpu.SemaphoreType.DMA,
        pltpu.SemaphoreType.DMA,
        pltpu.SemaphoreType.DMA,
        pltpu.SemaphoreType.DMA,
        pltpu.SemaphoreType.DMA,
    ],
    compiler_params=pltpu.CompilerParams(needs_layout_passes=False),
)
def _sampler_kernel(act_hbm, pidx_hbm, out_hbm, row_buf, idx_shared,
                    idx_a, idx_b, idx_c, out_a, out_b, out_c,
                    sem_act, sem_i0, sem_i1, sem_i2,
                    sem_o0, sem_o1, sem_o2):
    _sampler_body(act_hbm, pidx_hbm, out_hbm, row_buf, idx_shared,
                  (idx_a, idx_b, idx_c), (out_a, out_b, out_c),
                  sem_act, (sem_i0, sem_i1, sem_i2),
                  (sem_o0, sem_o1, sem_o2))


def kernel(activations, sample_indices):
    idx = sample_indices.astype(jnp.int32)
    packed = jnp.bitwise_or(idx[:, 0], jnp.left_shift(idx[:, 1], 16))
    return _sampler_kernel(activations, packed)
